# baseline (speedup 1.0000x reference)
import functools

import jax
import jax.numpy as jnp
from jax import lax
from jax.experimental import pallas as pl
from jax.experimental.pallas import tpu as pltpu
from jax.experimental.pallas import tpu_sc as plsc

N = 10000
E = 320000
D = 128

CB = 128
ROWS = (E + CB - 1) // CB
ROWS_PAD = ((ROWS + 255) // 256) * 256
E_PAD = ROWS_PAD * CB
NPAD = E_PAD - E
NTRASH = 16
NA = 10112
NB = NA // 128
IB = 8
EIB = 40
HALF = NA // 2
LBUF = 16 * HALF

_mesh = plsc.VectorSubcoreMesh(core_axis_name="c", subcore_axis_name="s")


@functools.partial(
    pl.kernel,
    out_type=jax.ShapeDtypeStruct((2, 16, NB, 128), jnp.float32),
    mesh=_mesh,
    scratch_types=[
        pltpu.VMEM((ROWS_PAD // 16, CB), jnp.int32),
        pltpu.VMEM((LBUF,), jnp.float32),
        pltpu.VMEM((NB, 128), jnp.float32),
    ],
    compiler_params=pltpu.CompilerParams(needs_layout_passes=False),
)
def _sc_hist(idx_hbm, hist_hbm, idxbuf, lhist, histloc):
    c = lax.axis_index("c")
    s = lax.axis_index("s")
    rpt = ROWS_PAD // 16
    lane = lax.iota(jnp.int32, 16) * HALF
    ones = jnp.ones((16,), jnp.float32)
    zero16 = jnp.zeros((16,), jnp.float32)
    pltpu.sync_copy(idx_hbm.at[c].at[pl.ds(s * rpt, rpt)], idxbuf)

    for p in range(2):
        base = p * HALF

        def zero(i, _):
            for k in range(16):
                lhist[pl.ds(i * 256 + k * 16, 16)] = zero16
            return 0

        lax.fori_loop(0, LBUF // 256, zero, 0)

        def scatter_row(r, _):
            for k in range(CB // 16):
                idxv = idxbuf[r, pl.ds(k * 16, 16)]
                inr = (idxv >= base) & (idxv < base + HALF)
                addr = lane + (idxv - base)
                plsc.addupdate_scatter(lhist, [addr], ones, mask=inr)
            return 0

        lax.fori_loop(0, rpt, scatter_row, 0)

        def drain(ci, _):
            flat = base + ci * 16
            acc = lhist[pl.ds(ci * 16, 16)]
            for l in range(1, 16):
                acc = acc + lhist[pl.ds(ci * 16 + l * HALF, 16)]
            histloc[flat // 128, pl.ds(flat % 128, 16)] = acc
            return 0

        lax.fori_loop(0, HALF // 16, drain, 0)

    pltpu.sync_copy(histloc, hist_hbm.at[c].at[s])


@functools.partial(
    pl.kernel,
    out_type=jax.ShapeDtypeStruct((2, NA, D), jnp.float32),
    mesh=_mesh,
    scratch_types=[
        pltpu.VMEM((EIB, CB), jnp.int32),
        pltpu.VMEM((EIB, CB), jnp.int32),
        pltpu.VMEM((CB, D), jnp.float32),
        pltpu.VMEM((CB, D), jnp.float32),
        pltpu.VMEM_SHARED((NA, D), jnp.float32),
        pltpu.SemaphoreType.DMA,
        pltpu.SemaphoreType.DMA,
        pltpu.SemaphoreType.DMA,
        pltpu.SemaphoreType.DMA,
    ],
)
def _sc_edges(xs_hbm, s_hbm, r_hbm, part_hbm,
              sbuf, rbuf, rows0, rows1, accsp, gsem0, gsem1, ssem0, ssem1):
    c = lax.axis_index("c")
    s = lax.axis_index("s")
    npt = NA // 16
    rpt = ROWS_PAD // 32
    base = c * (ROWS_PAD // 2) + s * rpt

    zero16 = jnp.zeros((16,), jnp.float32)

    def zrow(i, _):
        for k in range(D // 16):
            rows0[i, pl.ds(k * 16, 16)] = zero16
        return 0

    lax.fori_loop(0, CB, zrow, 0)
    for k in range(npt // CB):
        pltpu.sync_copy(rows0, accsp.at[pl.ds(s * npt + k * CB, CB)])
    rem = npt % CB
    if rem:
        pltpu.sync_copy(rows0.at[pl.ds(0, rem)],
                        accsp.at[pl.ds(s * npt + (npt // CB) * CB, rem)])
    plsc.subcore_barrier()

    rows = (rows0, rows1)
    gsem = (gsem0, gsem1)
    ssem = (ssem0, ssem1)

    def group(o, _):
        row0 = base + o * EIB
        pltpu.sync_copy(s_hbm.at[pl.ds(row0, EIB)], sbuf)
        pltpu.sync_copy(r_hbm.at[pl.ds(row0, EIB)], rbuf)
        pltpu.async_copy(xs_hbm.at[sbuf.at[0]], rows[0], gsem[0])
        for j in range(EIB):
            b, nb = j % 2, (j + 1) % 2
            if j + 1 < EIB:
                if j >= 1:
                    pltpu.make_async_copy(
                        rows[nb], accsp.at[rbuf.at[j - 1]], ssem[nb]).wait()
                pltpu.async_copy(xs_hbm.at[sbuf.at[j + 1]], rows[nb], gsem[nb])
            pltpu.make_async_copy(
                xs_hbm.at[sbuf.at[j]], rows[b], gsem[b]).wait()
            pltpu.async_copy(rows[b], accsp.at[rbuf.at[j]], ssem[b], add=True)
        pltpu.make_async_copy(
            rows[(EIB - 2) % 2], accsp.at[rbuf.at[EIB - 2]],
            ssem[(EIB - 2) % 2]).wait()
        pltpu.make_async_copy(
            rows[(EIB - 1) % 2], accsp.at[rbuf.at[EIB - 1]],
            ssem[(EIB - 1) % 2]).wait()
        return 0

    lax.fori_loop(0, rpt // EIB, group, 0)
    plsc.subcore_barrier()
    pltpu.sync_copy(accsp.at[pl.ds(s * npt, npt)],
                    part_hbm.at[c].at[pl.ds(s * npt, npt)])


def _mm_body(nodes_ref, wt_ref, b_ref, o_ref):
    o_ref[...] = jnp.dot(nodes_ref[...], wt_ref[...],
                         preferred_element_type=jnp.float32) + b_ref[...]


def _scale_body(y_ref, hs_ref, o_ref):
    deg = jnp.sum(hs_ref[0], axis=0) + 1.0
    o_ref[...] = y_ref[...] * lax.rsqrt(deg)[:, None]


def _final_body(p_ref, xs_ref, hr_ref, o_ref):
    t = p_ref[0] + p_ref[1] + xs_ref[...]
    rdeg = jnp.sum(hr_ref[0], axis=0) + 1.0
    t = t * lax.rsqrt(rdeg)[:, None]
    o_ref[...] = jnp.where(t >= 0.0, t, 0.01 * t)


def kernel(nodes, senders, receivers, W, b):
    senders = senders.astype(jnp.int32)
    receivers = receivers.astype(jnp.int32)
    trash = (N + (jnp.arange(NPAD, dtype=jnp.int32) % NTRASH))
    s_trash = jnp.concatenate([senders, trash])
    r_trash = jnp.concatenate([receivers, trash])
    idx_hist = jnp.stack([s_trash, r_trash]).reshape(2, ROWS_PAD, CB)
    s_gather = jnp.concatenate(
        [senders, (jnp.arange(NPAD, dtype=jnp.int32) % NTRASH)]
    ).reshape(ROWS_PAD, CB)
    r_gather = r_trash.reshape(ROWS_PAD, CB)

    hist = _sc_hist(idx_hist).reshape(2, 16, NA)

    wt = W.T
    b2 = b.reshape(1, D)
    y = pl.pallas_call(
        _mm_body,
        out_shape=jax.ShapeDtypeStruct((NA, D), jnp.float32),
        grid=(NB,),
        in_specs=[
            pl.BlockSpec((128, D), lambda i: (i, 0)),
            pl.BlockSpec((D, D), lambda i: (0, 0)),
            pl.BlockSpec((1, D), lambda i: (0, 0)),
        ],
        out_specs=pl.BlockSpec((128, D), lambda i: (i, 0)),
    )(nodes, wt, b2)
    xs = pl.pallas_call(
        _scale_body,
        out_shape=jax.ShapeDtypeStruct((NA, D), jnp.float32),
        grid=(NB,),
        in_specs=[
            pl.BlockSpec((128, D), lambda i: (i, 0)),
            pl.BlockSpec((1, 16, 128), lambda i: (0, 0, i)),
        ],
        out_specs=pl.BlockSpec((128, D), lambda i: (i, 0)),
    )(y, hist)

    part = _sc_edges(xs, s_gather, r_gather)

    out = pl.pallas_call(
        _final_body,
        out_shape=jax.ShapeDtypeStruct((N, D), jnp.float32),
        grid=(NB,),
        in_specs=[
            pl.BlockSpec((2, 128, D), lambda i: (0, i, 0)),
            pl.BlockSpec((128, D), lambda i: (i, 0)),
            pl.BlockSpec((1, 16, 128), lambda i: (1, 0, i)),
        ],
        out_specs=pl.BlockSpec((128, D), lambda i: (i, 0)),
    )(part, xs, hist)
    return out


# speedup vs baseline: 1.0098x; 1.0098x over previous
import functools

import jax
import jax.numpy as jnp
from jax import lax
from jax.experimental import pallas as pl
from jax.experimental.pallas import tpu as pltpu
from jax.experimental.pallas import tpu_sc as plsc

N = 10000
E = 320000
D = 128

CB = 128
ROWS = (E + CB - 1) // CB
ROWS_PAD = ((ROWS + 255) // 256) * 256
E_PAD = ROWS_PAD * CB
NPAD = E_PAD - E
NTRASH = 16
NA = 10112
NB = NA // 128
IB = 8
EIB = 40
HALF = NA // 2
LBUF = 16 * HALF

_mesh = plsc.VectorSubcoreMesh(core_axis_name="c", subcore_axis_name="s")


@functools.partial(
    pl.kernel,
    out_type=jax.ShapeDtypeStruct((2, 16, NB, 128), jnp.float32),
    mesh=_mesh,
    scratch_types=[
        pltpu.VMEM((ROWS_PAD // 16, CB), jnp.int32),
        pltpu.VMEM((LBUF,), jnp.float32),
        pltpu.VMEM((NB, 128), jnp.float32),
        pltpu.SemaphoreType.DMA,
    ],
    compiler_params=pltpu.CompilerParams(needs_layout_passes=False),
)
def _sc_hist(idx_hbm, hist_hbm, idxbuf, lhist, histloc, isem):
    c = lax.axis_index("c")
    s = lax.axis_index("s")
    rpt = ROWS_PAD // 16
    lane = lax.iota(jnp.int32, 16) * HALF
    ones = jnp.ones((16,), jnp.float32)
    zero16 = jnp.zeros((16,), jnp.float32)
    idx_src = idx_hbm.at[c].at[pl.ds(s * rpt, rpt)]
    pltpu.async_copy(idx_src, idxbuf, isem)

    for p in range(2):
        base = p * HALF

        def zero(i, _):
            for k in range(16):
                lhist[pl.ds(i * 256 + k * 16, 16)] = zero16
            return 0

        lax.fori_loop(0, LBUF // 256, zero, 0)
        if p == 0:
            pltpu.make_async_copy(idx_src, idxbuf, isem).wait()

        def scatter_row(r, _):
            for k in range(CB // 16):
                idxv = idxbuf[r, pl.ds(k * 16, 16)]
                inr = (idxv >= base) & (idxv < base + HALF)
                addr = lane + (idxv - base)
                plsc.addupdate_scatter(lhist, [addr], ones, mask=inr)
            return 0

        lax.fori_loop(0, rpt, scatter_row, 0)

        def drain(ci, _):
            flat = base + ci * 16
            acc = lhist[pl.ds(ci * 16, 16)]
            for l in range(1, 16):
                acc = acc + lhist[pl.ds(ci * 16 + l * HALF, 16)]
            histloc[flat // 128, pl.ds(flat % 128, 16)] = acc
            return 0

        lax.fori_loop(0, HALF // 16, drain, 0)

    pltpu.sync_copy(histloc, hist_hbm.at[c].at[s])


@functools.partial(
    pl.kernel,
    out_type=jax.ShapeDtypeStruct((2, NA, D), jnp.float32),
    mesh=_mesh,
    scratch_types=[
        pltpu.VMEM((EIB, CB), jnp.int32),
        pltpu.VMEM((EIB, CB), jnp.int32),
        pltpu.VMEM((CB, D), jnp.float32),
        pltpu.VMEM((CB, D), jnp.float32),
        pltpu.VMEM_SHARED((NA, D), jnp.float32),
        pltpu.SemaphoreType.DMA,
        pltpu.SemaphoreType.DMA,
        pltpu.SemaphoreType.DMA,
        pltpu.SemaphoreType.DMA,
    ],
)
def _sc_edges(xs_hbm, idx3_hbm, part_hbm,
              sbuf, rbuf, rows0, rows1, accsp, gsem0, gsem1, ssem0, ssem1):
    c = lax.axis_index("c")
    s = lax.axis_index("s")
    npt = NA // 16
    rpt = ROWS_PAD // 32
    base = c * (ROWS_PAD // 2) + s * rpt

    zero16 = jnp.zeros((16,), jnp.float32)

    def zrow(i, _):
        for k in range(D // 16):
            rows0[i, pl.ds(k * 16, 16)] = zero16
        return 0

    lax.fori_loop(0, CB, zrow, 0)
    for k in range(npt // CB):
        pltpu.sync_copy(rows0, accsp.at[pl.ds(s * npt + k * CB, CB)])
    rem = npt % CB
    if rem:
        pltpu.sync_copy(rows0.at[pl.ds(0, rem)],
                        accsp.at[pl.ds(s * npt + (npt // CB) * CB, rem)])
    plsc.subcore_barrier()

    rows = (rows0, rows1)
    gsem = (gsem0, gsem1)
    ssem = (ssem0, ssem1)

    def group(o, _):
        row0 = base + o * EIB
        pltpu.sync_copy(idx3_hbm.at[0].at[pl.ds(row0, EIB)], sbuf)
        pltpu.sync_copy(idx3_hbm.at[1].at[pl.ds(row0, EIB)], rbuf)
        pltpu.async_copy(xs_hbm.at[sbuf.at[0]], rows[0], gsem[0])
        for j in range(EIB):
            b, nb = j % 2, (j + 1) % 2
            if j + 1 < EIB:
                if j >= 1:
                    pltpu.make_async_copy(
                        rows[nb], accsp.at[rbuf.at[j - 1]], ssem[nb]).wait()
                pltpu.async_copy(xs_hbm.at[sbuf.at[j + 1]], rows[nb], gsem[nb])
            pltpu.make_async_copy(
                xs_hbm.at[sbuf.at[j]], rows[b], gsem[b]).wait()
            pltpu.async_copy(rows[b], accsp.at[rbuf.at[j]], ssem[b], add=True)
        pltpu.make_async_copy(
            rows[(EIB - 2) % 2], accsp.at[rbuf.at[EIB - 2]],
            ssem[(EIB - 2) % 2]).wait()
        pltpu.make_async_copy(
            rows[(EIB - 1) % 2], accsp.at[rbuf.at[EIB - 1]],
            ssem[(EIB - 1) % 2]).wait()
        return 0

    lax.fori_loop(0, rpt // EIB, group, 0)
    plsc.subcore_barrier()
    pltpu.sync_copy(accsp.at[pl.ds(s * npt, npt)],
                    part_hbm.at[c].at[pl.ds(s * npt, npt)])


def _mm_body(nodes_ref, wt_ref, b_ref, o_ref):
    o_ref[...] = jnp.dot(nodes_ref[...], wt_ref[...],
                         preferred_element_type=jnp.float32) + b_ref[...]


def _scale_body(y_ref, hs_ref, o_ref):
    deg = jnp.sum(hs_ref[0], axis=0) + 1.0
    o_ref[...] = y_ref[...] * lax.rsqrt(deg)[:, None]


def _final_body(p_ref, xs_ref, hr_ref, o_ref):
    t = p_ref[0] + p_ref[1] + xs_ref[...]
    rdeg = jnp.sum(hr_ref[0], axis=0) + 1.0
    t = t * lax.rsqrt(rdeg)[:, None]
    o_ref[...] = jnp.where(t >= 0.0, t, 0.01 * t)


def kernel(nodes, senders, receivers, W, b):
    senders = senders.astype(jnp.int32)
    receivers = receivers.astype(jnp.int32)
    trash = (N + (jnp.arange(NPAD, dtype=jnp.int32) % NTRASH))
    s_trash = jnp.concatenate([senders, trash])
    r_trash = jnp.concatenate([receivers, trash])
    idx_hist = jnp.stack([s_trash, r_trash]).reshape(2, ROWS_PAD, CB)

    hist = _sc_hist(idx_hist).reshape(2, 16, NA)

    wt = W.T
    b2 = b.reshape(1, D)
    y = pl.pallas_call(
        _mm_body,
        out_shape=jax.ShapeDtypeStruct((NA, D), jnp.float32),
        grid=(NB,),
        in_specs=[
            pl.BlockSpec((128, D), lambda i: (i, 0)),
            pl.BlockSpec((D, D), lambda i: (0, 0)),
            pl.BlockSpec((1, D), lambda i: (0, 0)),
        ],
        out_specs=pl.BlockSpec((128, D), lambda i: (i, 0)),
    )(nodes, wt, b2)
    xs = pl.pallas_call(
        _scale_body,
        out_shape=jax.ShapeDtypeStruct((NA, D), jnp.float32),
        grid=(NB,),
        in_specs=[
            pl.BlockSpec((128, D), lambda i: (i, 0)),
            pl.BlockSpec((1, 16, 128), lambda i: (0, 0, i)),
        ],
        out_specs=pl.BlockSpec((128, D), lambda i: (i, 0)),
    )(y, hist)

    part = _sc_edges(xs, idx_hist)

    out = pl.pallas_call(
        _final_body,
        out_shape=jax.ShapeDtypeStruct((N, D), jnp.float32),
        grid=(NB,),
        in_specs=[
            pl.BlockSpec((2, 128, D), lambda i: (0, i, 0)),
            pl.BlockSpec((128, D), lambda i: (i, 0)),
            pl.BlockSpec((1, 16, 128), lambda i: (1, 0, i)),
        ],
        out_specs=pl.BlockSpec((128, D), lambda i: (i, 0)),
    )(part, xs, hist)
    return out
